# baseline (speedup 1.0000x reference)
import functools

import jax
import jax.numpy as jnp
from jax import lax
from jax.experimental import pallas as pl
from jax.experimental.pallas import tpu as pltpu
from jax.experimental.pallas import tpu_sc as plsc

EMBED_DIM = 64
ROWS, COLS = 16384, 200
B = ROWS * COLS
NC, NS = 2, 16
NW = NC * NS
B_PER_W = B // NW
CHUNK = 512
N_CHUNK = B_PER_W // CHUNK

_mesh = plsc.VectorSubcoreMesh(core_axis_name="c", subcore_axis_name="s")


@functools.partial(
    pl.kernel,
    out_type=jax.ShapeDtypeStruct((B, EMBED_DIM), jnp.float32),
    mesh=_mesh,
    scratch_types=[
        pltpu.VMEM((CHUNK,), jnp.int32),
        pltpu.VMEM((CHUNK, EMBED_DIM), jnp.float32),
        pltpu.SemaphoreType.DMA,
    ],
    compiler_params=pltpu.CompilerParams(use_tc_tiling_on_sc=False),
)
def _gather(tok_hbm, table_hbm, out_hbm, idx_v, rows_v, sem):
    wid = lax.axis_index("s") * NC + lax.axis_index("c")
    base = wid * B_PER_W

    def body(i, carry):
        off = base + i * CHUNK
        pltpu.sync_copy(tok_hbm.at[pl.ds(off, CHUNK)], idx_v)
        pltpu.async_copy(table_hbm.at[idx_v], rows_v, sem).wait()
        pltpu.sync_copy(rows_v, out_hbm.at[pl.ds(off, CHUNK)])
        return carry

    lax.fori_loop(0, N_CHUNK, body, 0)


def kernel(tokens, table):
    flat = tokens.reshape(-1).astype(jnp.int32)
    out = _gather(flat, table)
    return out.reshape(ROWS, COLS, EMBED_DIM)


# speedup vs baseline: 1.0726x; 1.0726x over previous
import functools

import jax
import jax.numpy as jnp
from jax import lax
from jax.experimental import pallas as pl
from jax.experimental.pallas import tpu as pltpu
from jax.experimental.pallas import tpu_sc as plsc

EMBED_DIM = 64
ROWS, COLS = 16384, 200
B = ROWS * COLS
NC, NS = 2, 16
NW = NC * NS
B_PER_W = B // NW
CHUNK = 512
N_CHUNK = B_PER_W // CHUNK
NBUF = 2
N_GRP = N_CHUNK // NBUF

_mesh = plsc.VectorSubcoreMesh(core_axis_name="c", subcore_axis_name="s")


@functools.partial(
    pl.kernel,
    out_type=jax.ShapeDtypeStruct((B, EMBED_DIM), jnp.float32),
    mesh=_mesh,
    scratch_types=[
        pltpu.VMEM((CHUNK,), jnp.int32),
        pltpu.VMEM((CHUNK,), jnp.int32),
        pltpu.VMEM((CHUNK, EMBED_DIM), jnp.float32),
        pltpu.VMEM((CHUNK, EMBED_DIM), jnp.float32),
        pltpu.SemaphoreType.DMA,
        pltpu.SemaphoreType.DMA,
        pltpu.SemaphoreType.DMA,
        pltpu.SemaphoreType.DMA,
        pltpu.SemaphoreType.DMA,
    ],
    compiler_params=pltpu.CompilerParams(use_tc_tiling_on_sc=False),
)
def _gather(tok_hbm, table_hbm, out_hbm,
            idx0, idx1, rows0, rows1, si0, si1, sw0, sw1, sg):
    idx = (idx0, idx1)
    rows = (rows0, rows1)
    si = (si0, si1)
    sw = (sw0, sw1)

    wid = lax.axis_index("s") * NC + lax.axis_index("c")
    base = wid * B_PER_W
    max_off = B - CHUNK

    def issue_idx(b, i):
        off = jnp.minimum(base + i * CHUNK, max_off)
        pltpu.async_copy(tok_hbm.at[pl.ds(off, CHUNK)], idx[b], si[b])

    def wait_idx(b):
        pltpu.make_async_copy(tok_hbm.at[pl.ds(0, CHUNK)], idx[b], si[b]).wait()

    def wait_wb(b):
        pltpu.make_async_copy(rows[b], out_hbm.at[pl.ds(0, CHUNK)], sw[b]).wait()

    def gather(b):
        pltpu.async_copy(table_hbm.at[idx[b]], rows[b], sg).wait()

    def issue_wb(b, i):
        off = base + i * CHUNK
        pltpu.async_copy(rows[b], out_hbm.at[pl.ds(off, CHUNK)], sw[b])

    for b in range(NBUF):
        issue_idx(b, b)

    for i in range(NBUF):
        wait_idx(i)
        gather(i)
        issue_wb(i, i)
        issue_idx(i, i + NBUF)

    def body(g, carry):
        for b in range(NBUF):
            i = g * NBUF + b
            wait_idx(b)
            wait_wb(b)
            gather(b)
            issue_wb(b, i)
            issue_idx(b, i + NBUF)
        return carry

    lax.fori_loop(1, N_GRP, body, 0)

    for b in range(NBUF):
        wait_wb(b)
        wait_idx(b)


def kernel(tokens, table):
    flat = tokens.reshape(-1).astype(jnp.int32)
    out = _gather(flat, table)
    return out.reshape(ROWS, COLS, EMBED_DIM)


# speedup vs baseline: 1.0809x; 1.0077x over previous
import functools

import jax
import jax.numpy as jnp
from jax import lax
from jax.experimental import pallas as pl
from jax.experimental.pallas import tpu as pltpu
from jax.experimental.pallas import tpu_sc as plsc

EMBED_DIM = 64
ROWS, COLS = 16384, 200
B = ROWS * COLS
NC, NS = 2, 16
NW = NC * NS
B_PER_W = B // NW
CHUNK = 512
N_CHUNK = B_PER_W // CHUNK
NBUF = 3
HEAD = 5
TAIL = 3
N_GRP = (N_CHUNK - HEAD - TAIL) // NBUF

_mesh = plsc.VectorSubcoreMesh(core_axis_name="c", subcore_axis_name="s")


@functools.partial(
    pl.kernel,
    out_type=jax.ShapeDtypeStruct((B, EMBED_DIM), jnp.float32),
    mesh=_mesh,
    scratch_types=[
        pltpu.VMEM((CHUNK,), jnp.int32),
        pltpu.VMEM((CHUNK,), jnp.int32),
        pltpu.VMEM((CHUNK,), jnp.int32),
        pltpu.VMEM((CHUNK, EMBED_DIM), jnp.float32),
        pltpu.VMEM((CHUNK, EMBED_DIM), jnp.float32),
        pltpu.VMEM((CHUNK, EMBED_DIM), jnp.float32),
        pltpu.SemaphoreType.DMA,
        pltpu.SemaphoreType.DMA,
        pltpu.SemaphoreType.DMA,
        pltpu.SemaphoreType.DMA,
        pltpu.SemaphoreType.DMA,
        pltpu.SemaphoreType.DMA,
        pltpu.SemaphoreType.DMA,
        pltpu.SemaphoreType.DMA,
        pltpu.SemaphoreType.DMA,
    ],
    compiler_params=pltpu.CompilerParams(use_tc_tiling_on_sc=False),
)
def _gather(tok_hbm, table_hbm, out_hbm,
            idx0, idx1, idx2, rows0, rows1, rows2,
            si0, si1, si2, sg0, sg1, sg2, sw0, sw1, sw2):
    idx = (idx0, idx1, idx2)
    rows = (rows0, rows1, rows2)
    si = (si0, si1, si2)
    sg = (sg0, sg1, sg2)
    sw = (sw0, sw1, sw2)

    wid = lax.axis_index("s") * NC + lax.axis_index("c")
    base = wid * B_PER_W

    def issue_idx(b, i):
        pltpu.async_copy(tok_hbm.at[pl.ds(base + i * CHUNK, CHUNK)], idx[b], si[b])

    def wait_idx(b):
        pltpu.make_async_copy(tok_hbm.at[pl.ds(0, CHUNK)], idx[b], si[b]).wait()

    def fire_gather(b):
        pltpu.async_copy(table_hbm.at[idx[b]], rows[b], sg[b])

    def wait_gather(b):
        pltpu.make_async_copy(table_hbm.at[idx[b]], rows[b], sg[b]).wait()

    def issue_wb(b, i):
        pltpu.async_copy(rows[b], out_hbm.at[pl.ds(base + i * CHUNK, CHUNK)], sw[b])

    def wait_wb(b):
        pltpu.make_async_copy(rows[b], out_hbm.at[pl.ds(0, CHUNK)], sw[b]).wait()

    def step(i, b, bj, need_wb_wait, has_prev, do_prefetch):
        wait_idx(b)
        if need_wb_wait:
            wait_wb(b)
        fire_gather(b)
        if has_prev:
            wait_gather(bj)
            issue_wb(bj, i - 1)
            if do_prefetch:
                issue_idx(bj, i + NBUF - 1)

    for b in range(NBUF):
        issue_idx(b, b)

    for i in range(HEAD):
        step(i, i % NBUF, (i - 1) % NBUF, i >= NBUF, i >= 1, True)

    def body(g, carry):
        for u in range(NBUF):
            i = HEAD + g * NBUF + u
            step(i, (HEAD + u) % NBUF, (HEAD + u - 1) % NBUF, True, True, True)
        return carry

    lax.fori_loop(0, N_GRP, body, 0)

    for i in range(N_CHUNK - TAIL, N_CHUNK):
        step(i, i % NBUF, (i - 1) % NBUF, True, True, i + NBUF - 1 < N_CHUNK)

    last = N_CHUNK - 1
    wait_gather(last % NBUF)
    issue_wb(last % NBUF, last)
    for i in range(N_CHUNK - NBUF, N_CHUNK):
        wait_wb(i % NBUF)


def kernel(tokens, table):
    flat = tokens.reshape(-1).astype(jnp.int32)
    out = _gather(flat, table)
    return out.reshape(ROWS, COLS, EMBED_DIM)
